# baseline (speedup 1.0000x reference)
import jax
import jax.numpy as jnp
from jax import lax
from jax.experimental import pallas as pl
from jax.experimental.pallas import tpu as pltpu
from jax.experimental.pallas import tpu_sc as plsc

N = 10000
D = 128
_SC_ROWS = 112
_TC_ROWS = N - _SC_ROWS
_BM = 96
_KSLICE = 640
_KSMALL = 512
_KCOV = 14 * _KSLICE + 2 * _KSMALL
_RPC = _SC_ROWS // 2
_GROUP = 4


def _tc_mm_kernel(adj_ref, h_ref, out_ref):
    out_ref[...] = jnp.dot(adj_ref[...], h_ref[...],
                           preferred_element_type=jnp.float32)


def _sc_partials_kernel(adj_hbm, h_hbm, out_hbm, adj_v, h_v, stage_v, sem):
    c = lax.axis_index("c")
    s = lax.axis_index("s")
    row0 = pl.multiple_of(_TC_ROWS + c * _RPC, 8)

    @pl.when(s < 14)
    def _load_main():
        k0 = pl.multiple_of(s * _KSLICE, 128)
        cp_a = pltpu.make_async_copy(
            adj_hbm.at[pl.ds(row0, _RPC), pl.ds(k0, _KSLICE)], adj_v, sem)
        cp_h = pltpu.make_async_copy(h_hbm.at[pl.ds(k0, _KSLICE)], h_v, sem)
        cp_a.start()
        cp_h.start()
        cp_a.wait()
        cp_h.wait()

    @pl.when(s >= 14)
    def _load_small():
        k0 = pl.multiple_of(14 * _KSLICE + (s - 14) * _KSMALL, 128)
        cp_a = pltpu.make_async_copy(
            adj_hbm.at[pl.ds(row0, _RPC), pl.ds(k0, _KSMALL)],
            adj_v.at[:, pl.ds(0, _KSMALL)], sem)
        cp_h = pltpu.make_async_copy(
            h_hbm.at[pl.ds(k0, _KSMALL)], h_v.at[pl.ds(0, _KSMALL)], sem)
        cp_a.start()
        cp_h.start()
        cp_a.wait()
        cp_h.wait()

    nblk = jnp.where(s >= 14, _KSMALL // 16, _KSLICE // 16)
    zero = jnp.zeros((16,), jnp.float32)
    for g in range(_RPC // _GROUP):
        def body(k16, accs):
            base = k16 * 16
            av = [adj_v[g * _GROUP + r, pl.ds(base, 16)]
                  for r in range(_GROUP)]
            accs = list(accs)
            for kk in range(16):
                hv = [h_v[base + kk, pl.ds(16 * j, 16)] for j in range(8)]
                for r in range(_GROUP):
                    a = av[r][kk]
                    for j in range(8):
                        accs[r * 8 + j] = accs[r * 8 + j] + a * hv[j]
            return tuple(accs)

        accs = lax.fori_loop(0, nblk, body, tuple([zero] * (_GROUP * 8)))
        half = (g % 2) * _GROUP
        for r in range(_GROUP):
            for j in range(8):
                stage_v[half + r, pl.ds(16 * j, 16)] = accs[r * 8 + j]
        if g % 2 == 1:
            pltpu.sync_copy(
                stage_v,
                out_hbm.at[s, pl.ds(c * _RPC + (g - 1) * _GROUP, 2 * _GROUP)])


def _sc_partials(adj, h):
    mesh = plsc.VectorSubcoreMesh(core_axis_name="c", subcore_axis_name="s")
    return pl.kernel(
        _sc_partials_kernel,
        out_type=jax.ShapeDtypeStruct((16, _SC_ROWS, D), jnp.float32),
        mesh=mesh,
        scratch_types=[
            pltpu.VMEM((_RPC, _KSLICE), jnp.float32),
            pltpu.VMEM((_KSLICE, D), jnp.float32),
            pltpu.VMEM((2 * _GROUP, D), jnp.float32),
            pltpu.SemaphoreType.DMA,
        ],
    )(adj, h)


def _combine_kernel(tc_ref, part_ref, adj_tail_ref, h_tail_ref, out_ref):
    del tc_ref
    out_ref[...] = (jnp.sum(part_ref[...], axis=0)
                    + jnp.dot(adj_tail_ref[...], h_tail_ref[...],
                              preferred_element_type=jnp.float32))


def kernel(adj, h):
    partials = _sc_partials(adj, h)

    tc_out = pl.pallas_call(
        _tc_mm_kernel,
        grid=(_TC_ROWS // _BM,),
        in_specs=[
            pl.BlockSpec((_BM, N), lambda i: (i, 0)),
            pl.BlockSpec((N, D), lambda i: (0, 0)),
        ],
        out_specs=pl.BlockSpec((_BM, D), lambda i: (i, 0)),
        out_shape=jax.ShapeDtypeStruct((N, D), jnp.float32),
        compiler_params=pltpu.CompilerParams(
            dimension_semantics=("parallel",)),
    )(adj, h)

    adj_tail = lax.slice(adj, (_TC_ROWS, _KCOV), (N, N))
    h_tail = lax.slice(h, (_KCOV, 0), (N, D))

    nblk = _SC_ROWS // 8
    first = _TC_ROWS // 8
    return pl.pallas_call(
        _combine_kernel,
        grid=(nblk,),
        in_specs=[
            pl.BlockSpec(memory_space=pl.MemorySpace.ANY),
            pl.BlockSpec((16, 8, D), lambda i: (0, i, 0)),
            pl.BlockSpec((8, 16), lambda i: (i, 0)),
            pl.BlockSpec((16, D), lambda i: (0, 0)),
        ],
        out_specs=pl.BlockSpec((8, D), lambda i: (first + i, 0)),
        out_shape=jax.ShapeDtypeStruct((N, D), jnp.float32),
        input_output_aliases={0: 0},
    )(tc_out, partials, adj_tail, h_tail)


# speedup vs baseline: 1.4924x; 1.4924x over previous
import jax
import jax.numpy as jnp
from jax.experimental import pallas as pl
from jax.experimental.pallas import tpu as pltpu

_BM = 200


def _mm_kernel(adj_ref, h_ref, out_ref):
    out_ref[...] = jnp.dot(adj_ref[...], h_ref[...],
                           preferred_element_type=jnp.float32)


def kernel(adj, h):
    n, k = adj.shape
    d = h.shape[1]
    grid = (n // _BM,)
    return pl.pallas_call(
        _mm_kernel,
        grid=grid,
        in_specs=[
            pl.BlockSpec((_BM, k), lambda i: (i, 0)),
            pl.BlockSpec((k, d), lambda i: (0, 0)),
        ],
        out_specs=pl.BlockSpec((_BM, d), lambda i: (i, 0)),
        out_shape=jax.ShapeDtypeStruct((n, d), jnp.float32),
        compiler_params=pltpu.CompilerParams(
            dimension_semantics=("parallel",)),
    )(adj, h)
